# baseline (speedup 1.0000x reference)
import functools

import jax
import jax.numpy as jnp
import numpy as np
from jax import lax
from jax.experimental import pallas as pl
from jax.experimental.pallas import tpu as pltpu
from jax.experimental.pallas import tpu_sc as plsc

TOP_P = 0.9
B = 128
V = 100000
VPAD = 100352
NTILES = 16
CHUNK = VPAD // NTILES
NVEC = CHUNK // 16
NJ = CHUNK // 128
HALF0 = 3200
NJ0 = HALF0 // 128
NJ1 = (CHUNK - HALF0) // 128
ROWS_PER_CORE = B // 2

_MIN32 = np.int32(-2147483648)


def _digit(kvec_i32, shift):
    ku = plsc.bitcast(kvec_i32, jnp.uint32)
    d = (ku >> jnp.uint32(shift)) & jnp.uint32(255)
    return d.astype(jnp.int32)


def _sc_sort_body(x_hbm, keys_hbm, order_hbm,
                  ak, ai, bk, bi, hg,
                  xw, kw, iw, dw, dvm, hist, offs, hgv, sem):
    c = lax.axis_index("c")
    s = lax.axis_index("s")
    base = s * CHUNK
    lane = lax.iota(jnp.int32, 16)
    ones16 = jnp.ones((16,), jnp.int32)
    zeros16 = jnp.zeros((16,), jnp.int32)

    def row_body2(i, _):
        row = c * ROWS_PER_CORE + i

        def one_pass2(src_k, src_i, dst_k, dst_i, shift, first):
            if first:
                pltpu.sync_copy(x_hbm.at[row, pl.ds(base, CHUNK)], xw)

                def xf(j, _):
                    for t in range(8):
                        off = j * 128 + t * 16
                        xv = xw[pl.ds(off, 16)]
                        u = plsc.bitcast(xv, jnp.int32)
                        m = jnp.where(u < 0, ~u, u ^ _MIN32)
                        kw[pl.ds(off, 16)] = ~m
                        iw[pl.ds(off, 16)] = base + off + lane
                    return _
                lax.fori_loop(0, NJ, xf, 0, unroll=False)
            else:
                pltpu.sync_copy(src_k.at[pl.ds(base, CHUNK)], kw)
                pltpu.sync_copy(src_i.at[pl.ds(base, CHUNK)], iw)

            def hz(j, _):
                hist[pl.ds(j * 16, 16)] = zeros16
                return _
            lax.fori_loop(0, 64, hz, 0, unroll=True)

            def hloop(j, _):
                for t in range(8):
                    salt = jnp.int32((t & 1) * 512)
                    off0 = j * 128 + t * 16
                    d0 = _digit(kw[pl.ds(off0, 16)], shift)
                    dvm[pl.ds(off0, 16)] = d0
                    plsc.addupdate_scatter(hist, [d0 + salt], ones16)
                    off1 = HALF0 + j * 128 + t * 16
                    d1 = _digit(kw[pl.ds(off1, 16)], shift) + jnp.int32(256)
                    dvm[pl.ds(off1, 16)] = d1
                    plsc.addupdate_scatter(hist, [d1 + salt], ones16)
                return _
            lax.fori_loop(0, NJ1, hloop, 0, unroll=False)

            def htail(j, _):
                for t in range(8):
                    salt = jnp.int32((t & 1) * 512)
                    off0 = j * 128 + t * 16
                    d0 = _digit(kw[pl.ds(off0, 16)], shift)
                    dvm[pl.ds(off0, 16)] = d0
                    plsc.addupdate_scatter(hist, [d0 + salt], ones16)
                return _
            lax.fori_loop(NJ1, NJ0, htail, 0, unroll=False)

            def hsum_loop(j, _):
                hist[pl.ds(j * 16, 16)] = (hist[pl.ds(j * 16, 16)]
                                           + hist[pl.ds(512 + j * 16, 16)])
                return _
            lax.fori_loop(0, 32, hsum_loop, 0, unroll=True)

            pltpu.sync_copy(hist.at[pl.ds(0, 256)], hg.at[2 * s])
            pltpu.sync_copy(hist.at[pl.ds(256, 256)], hg.at[2 * s + 1])
            plsc.subcore_barrier()

            pltpu.sync_copy(hg, hgv)
            carry = jnp.int32(0)
            for jb in range(16):
                tot = zeros16
                pri0 = zeros16
                for t in range(2 * NTILES):
                    rowv = hgv[t, pl.ds(jb * 16, 16)]
                    tot = tot + rowv
                    pri0 = pri0 + jnp.where(jnp.int32(t) < 2 * s, rowv,
                                            zeros16)
                pri1 = pri0 + hgv[2 * s, pl.ds(jb * 16, 16)]
                csum = plsc.cumsum(tot)
                excl = carry + (csum - tot)
                offs[pl.ds(jb * 16, 16)] = excl + pri0
                offs[pl.ds(256 + jb * 16, 16)] = excl + pri1
                carry = carry + jnp.sum(tot)

            def dpair(j, dst_k, dst_i, shift):
                for t in range(8):
                    off0 = j * 128 + t * 16
                    off1 = HALF0 + j * 128 + t * 16
                    d0 = dvm[pl.ds(off0, 16)]
                    d1 = dvm[pl.ds(off1, 16)]
                    occ0, last0 = plsc.scan_count(d0)
                    occ1, last1 = plsc.scan_count(d1)
                    b0 = plsc.load_gather(offs, [d0])
                    b1 = plsc.load_gather(offs, [d1])
                    dw[j, pl.ds(t * 16, 16)] = b0 + occ0 - 1
                    dw[NJ0 + j, pl.ds(t * 16, 16)] = b1 + occ1 - 1
                    plsc.store_scatter(offs, [d0], b0 + occ0, mask=last0)
                    plsc.store_scatter(offs, [d1], b1 + occ1, mask=last1)
                pltpu.make_async_copy(
                    kw.at[pl.ds(j * 128, 128)], dst_k.at[dw.at[j]], sem
                ).start()
                pltpu.make_async_copy(
                    iw.at[pl.ds(j * 128, 128)], dst_i.at[dw.at[j]], sem
                ).start()
                pltpu.make_async_copy(
                    kw.at[pl.ds(HALF0 + j * 128, 128)],
                    dst_k.at[dw.at[NJ0 + j]], sem
                ).start()
                pltpu.make_async_copy(
                    iw.at[pl.ds(HALF0 + j * 128, 128)],
                    dst_i.at[dw.at[NJ0 + j]], sem
                ).start()

            def dloop(j, _):
                dpair(j, dst_k, dst_i, shift)
                return _
            lax.fori_loop(0, NJ1, dloop, 0, unroll=False)

            def dtail(j, _):
                for t in range(8):
                    off0 = j * 128 + t * 16
                    d0 = dvm[pl.ds(off0, 16)]
                    occ0, last0 = plsc.scan_count(d0)
                    b0 = plsc.load_gather(offs, [d0])
                    dw[j, pl.ds(t * 16, 16)] = b0 + occ0 - 1
                    plsc.store_scatter(offs, [d0], b0 + occ0, mask=last0)
                pltpu.make_async_copy(
                    kw.at[pl.ds(j * 128, 128)], dst_k.at[dw.at[j]], sem
                ).start()
                pltpu.make_async_copy(
                    iw.at[pl.ds(j * 128, 128)], dst_i.at[dw.at[j]], sem
                ).start()
                return _
            lax.fori_loop(NJ1, NJ0, dtail, 0, unroll=False)

            pltpu.make_async_copy(
                x_hbm.at[row, pl.ds(base, CHUNK)], xw, sem).wait()
            pltpu.make_async_copy(
                x_hbm.at[row, pl.ds(base, CHUNK)], xw, sem).wait()
            plsc.subcore_barrier()

        one_pass2(None, None, ak, ai, 0, True)
        one_pass2(ak, ai, bk, bi, 8, False)
        one_pass2(bk, bi, ak, ai, 16, False)
        one_pass2(ak, ai, bk, bi, 24, False)
        pltpu.sync_copy(bk.at[pl.ds(base, CHUNK)],
                        keys_hbm.at[row, pl.ds(base, CHUNK)])
        pltpu.sync_copy(bi.at[pl.ds(base, CHUNK)],
                        order_hbm.at[row, pl.ds(base, CHUNK)])
        return _

    lax.fori_loop(0, ROWS_PER_CORE, row_body2, 0, unroll=False)


def _sc_sort(xpad):
    mesh = plsc.VectorSubcoreMesh(core_axis_name="c", subcore_axis_name="s")
    f = pl.kernel(
        _sc_sort_body,
        mesh=mesh,
        compiler_params=pltpu.CompilerParams(needs_layout_passes=False),
        out_type=[
            jax.ShapeDtypeStruct((B, VPAD), jnp.int32),
            jax.ShapeDtypeStruct((B, VPAD), jnp.int32),
        ],
        scratch_types=[
            pltpu.VMEM_SHARED((VPAD,), jnp.int32),
            pltpu.VMEM_SHARED((VPAD,), jnp.int32),
            pltpu.VMEM_SHARED((VPAD,), jnp.int32),
            pltpu.VMEM_SHARED((VPAD,), jnp.int32),
            pltpu.VMEM_SHARED((2 * NTILES, 256), jnp.int32),
            pltpu.VMEM((CHUNK,), jnp.float32),
            pltpu.VMEM((CHUNK,), jnp.int32),
            pltpu.VMEM((CHUNK,), jnp.int32),
            pltpu.VMEM((NJ, 128), jnp.int32),
            pltpu.VMEM((CHUNK,), jnp.int32),
            pltpu.VMEM((1024,), jnp.int32),
            pltpu.VMEM((512,), jnp.int32),
            pltpu.VMEM((2 * NTILES, 256), jnp.int32),
            pltpu.SemaphoreType.DMA,
        ],
    )
    return f(xpad)


ROWS_PER_BLOCK = 8


def _tail_kernel(keys_ref, order_ref, gum_ref, tri_ref, tri2_ref, out_ref):
    k = keys_ref[...]
    m = ~k
    u = jnp.where(m < 0, m ^ _MIN32, ~m)
    s = lax.bitcast_convert_type(u, jnp.float32)
    maxv = s[:, 0:1]
    e = jnp.exp(s - maxv)
    z = jnp.sum(e, axis=-1, keepdims=True)
    p = e / z
    p3 = p.reshape(ROWS_PER_BLOCK * (VPAD // 128), 128)
    within_excl = jnp.dot(p3, tri_ref[...],
                          preferred_element_type=jnp.float32)
    bsum = jnp.sum(p3, axis=-1).reshape(ROWS_PER_BLOCK, VPAD // 128)
    bcarry = jnp.dot(bsum, tri2_ref[...],
                     preferred_element_type=jnp.float32)
    excl = (within_excl.reshape(ROWS_PER_BLOCK, VPAD // 128, 128)
            + bcarry[:, :, None]).reshape(ROWS_PER_BLOCK, VPAD)
    keep = excl <= jnp.float32(TOP_P)
    score = jnp.where(keep, s + gum_ref[...], -jnp.inf)
    jstar = jnp.argmax(score, axis=-1, keepdims=True)
    cols = lax.broadcasted_iota(jnp.int32, (ROWS_PER_BLOCK, VPAD), 1)
    sel = jnp.where(cols == jstar, order_ref[...], jnp.int32(-1))
    out_ref[...] = jnp.max(sel, axis=-1, keepdims=True)


def _tail(keys, order, gumbel, tri, tri2):
    grid = (B // ROWS_PER_BLOCK,)
    return pl.pallas_call(
        _tail_kernel,
        grid=grid,
        in_specs=[
            pl.BlockSpec((ROWS_PER_BLOCK, VPAD), lambda i: (i, 0)),
            pl.BlockSpec((ROWS_PER_BLOCK, VPAD), lambda i: (i, 0)),
            pl.BlockSpec((ROWS_PER_BLOCK, VPAD), lambda i: (i, 0)),
            pl.BlockSpec((128, 128), lambda i: (0, 0)),
            pl.BlockSpec((VPAD // 128, VPAD // 128), lambda i: (0, 0)),
        ],
        out_specs=pl.BlockSpec((ROWS_PER_BLOCK, 1), lambda i: (i, 0)),
        out_shape=jax.ShapeDtypeStruct((B, 1), jnp.int32),
    )(keys, order, gumbel, tri, tri2)


def kernel(logits):
    x = logits / 1.0
    xpad = jnp.pad(x, ((0, 0), (0, VPAD - V)), constant_values=-jnp.inf)
    keys, order = _sc_sort(xpad)

    skey = jax.random.key(42)
    u = jax.random.uniform(skey, (B, V), dtype=jnp.float32,
                           minval=1e-20, maxval=1.0)
    gumbel = -jnp.log(-jnp.log(u))
    gumbel_p = jnp.pad(gumbel, ((0, 0), (0, VPAD - V)))

    tri = jnp.triu(jnp.ones((128, 128), jnp.float32), k=1)
    n2 = VPAD // 128
    tri2 = jnp.triu(jnp.ones((n2, n2), jnp.float32), k=1)

    return _tail(keys, order, gumbel_p, tri, tri2)


# speedup vs baseline: 1.0150x; 1.0150x over previous
import functools

import jax
import jax.numpy as jnp
import numpy as np
from jax import lax
from jax.experimental import pallas as pl
from jax.experimental.pallas import tpu as pltpu
from jax.experimental.pallas import tpu_sc as plsc

TOP_P = 0.9
B = 128
V = 100000
VPAD = 100352
NTILES = 16
CHUNK = VPAD // NTILES
NVEC = CHUNK // 16
NJ = CHUNK // 128
HALF0 = 3200
NJ0 = HALF0 // 128
NJ1 = (CHUNK - HALF0) // 128
ROWS_PER_CORE = B // 2

_MIN32 = np.int32(-2147483648)


def _digit(kvec_i32, shift):
    ku = plsc.bitcast(kvec_i32, jnp.uint32)
    d = (ku >> jnp.uint32(shift)) & jnp.uint32(255)
    return d.astype(jnp.int32)


def _sc_sort_body(x_hbm, keys_hbm, order_hbm,
                  ak, ai, bk, bi, hg,
                  xw, kw, iw, dw, hist, offs, hgv, sem):
    c = lax.axis_index("c")
    s = lax.axis_index("s")
    base = s * CHUNK
    lane = lax.iota(jnp.int32, 16)
    ones16 = jnp.ones((16,), jnp.int32)
    zeros16 = jnp.zeros((16,), jnp.int32)

    def row_body2(i, _):
        row = c * ROWS_PER_CORE + i

        def one_pass2(src_k, src_i, dst_k, dst_i, shift, first):
            if first:
                pltpu.sync_copy(x_hbm.at[row, pl.ds(base, CHUNK)], xw)

                def xf(j, _):
                    for t in range(8):
                        off = j * 128 + t * 16
                        xv = xw[pl.ds(off, 16)]
                        u = plsc.bitcast(xv, jnp.int32)
                        m = jnp.where(u < 0, ~u, u ^ _MIN32)
                        kw[pl.ds(off, 16)] = ~m
                        iw[pl.ds(off, 16)] = base + off + lane
                    return _
                lax.fori_loop(0, NJ, xf, 0, unroll=False)
            else:
                pltpu.sync_copy(src_k.at[pl.ds(base, CHUNK)], kw)
                pltpu.sync_copy(src_i.at[pl.ds(base, CHUNK)], iw)

            def hz(j, _):
                hist[pl.ds(j * 16, 16)] = zeros16
                return _
            lax.fori_loop(0, 32, hz, 0, unroll=True)

            def hloop(j, _):
                for t in range(8):
                    off0 = j * 128 + t * 16
                    d0 = _digit(kw[pl.ds(off0, 16)], shift)
                    plsc.addupdate_scatter(hist, [d0], ones16)
                    off1 = HALF0 + j * 128 + t * 16
                    d1 = _digit(kw[pl.ds(off1, 16)], shift) + jnp.int32(256)
                    plsc.addupdate_scatter(hist, [d1], ones16)
                return _
            lax.fori_loop(0, NJ1, hloop, 0, unroll=False)

            def htail(j, _):
                for t in range(8):
                    off0 = j * 128 + t * 16
                    d0 = _digit(kw[pl.ds(off0, 16)], shift)
                    plsc.addupdate_scatter(hist, [d0], ones16)
                return _
            lax.fori_loop(NJ1, NJ0, htail, 0, unroll=False)

            pltpu.sync_copy(hist.at[pl.ds(0, 256)], hg.at[2 * s])
            pltpu.sync_copy(hist.at[pl.ds(256, 256)], hg.at[2 * s + 1])
            plsc.subcore_barrier()

            pltpu.sync_copy(hg, hgv)
            carry = jnp.int32(0)
            for jb in range(16):
                tot = zeros16
                pri0 = zeros16
                for t in range(2 * NTILES):
                    rowv = hgv[t, pl.ds(jb * 16, 16)]
                    tot = tot + rowv
                    pri0 = pri0 + jnp.where(jnp.int32(t) < 2 * s, rowv,
                                            zeros16)
                pri1 = pri0 + hgv[2 * s, pl.ds(jb * 16, 16)]
                csum = plsc.cumsum(tot)
                excl = carry + (csum - tot)
                offs[pl.ds(jb * 16, 16)] = excl + pri0
                offs[pl.ds(256 + jb * 16, 16)] = excl + pri1
                carry = carry + jnp.sum(tot)

            def dpair(j, dst_k, dst_i, shift):
                for t in range(8):
                    off0 = j * 128 + t * 16
                    off1 = HALF0 + j * 128 + t * 16
                    d0 = _digit(kw[pl.ds(off0, 16)], shift)
                    d1 = _digit(kw[pl.ds(off1, 16)], shift) + jnp.int32(256)
                    occ0, last0 = plsc.scan_count(d0)
                    occ1, last1 = plsc.scan_count(d1)
                    b0 = plsc.load_gather(offs, [d0])
                    b1 = plsc.load_gather(offs, [d1])
                    dw[j, pl.ds(t * 16, 16)] = b0 + occ0 - 1
                    dw[NJ0 + j, pl.ds(t * 16, 16)] = b1 + occ1 - 1
                    plsc.store_scatter(offs, [d0], b0 + occ0, mask=last0)
                    plsc.store_scatter(offs, [d1], b1 + occ1, mask=last1)
                pltpu.make_async_copy(
                    kw.at[pl.ds(j * 128, 128)], dst_k.at[dw.at[j]], sem
                ).start()
                pltpu.make_async_copy(
                    iw.at[pl.ds(j * 128, 128)], dst_i.at[dw.at[j]], sem
                ).start()
                pltpu.make_async_copy(
                    kw.at[pl.ds(HALF0 + j * 128, 128)],
                    dst_k.at[dw.at[NJ0 + j]], sem
                ).start()
                pltpu.make_async_copy(
                    iw.at[pl.ds(HALF0 + j * 128, 128)],
                    dst_i.at[dw.at[NJ0 + j]], sem
                ).start()

            def dloop(j, _):
                dpair(j, dst_k, dst_i, shift)
                return _
            lax.fori_loop(0, NJ1, dloop, 0, unroll=False)

            def dtail(j, _):
                for t in range(8):
                    off0 = j * 128 + t * 16
                    d0 = _digit(kw[pl.ds(off0, 16)], shift)
                    occ0, last0 = plsc.scan_count(d0)
                    b0 = plsc.load_gather(offs, [d0])
                    dw[j, pl.ds(t * 16, 16)] = b0 + occ0 - 1
                    plsc.store_scatter(offs, [d0], b0 + occ0, mask=last0)
                pltpu.make_async_copy(
                    kw.at[pl.ds(j * 128, 128)], dst_k.at[dw.at[j]], sem
                ).start()
                pltpu.make_async_copy(
                    iw.at[pl.ds(j * 128, 128)], dst_i.at[dw.at[j]], sem
                ).start()
                return _
            lax.fori_loop(NJ1, NJ0, dtail, 0, unroll=False)

            pltpu.make_async_copy(
                x_hbm.at[row, pl.ds(base, CHUNK)], xw, sem).wait()
            pltpu.make_async_copy(
                x_hbm.at[row, pl.ds(base, CHUNK)], xw, sem).wait()
            plsc.subcore_barrier()

        one_pass2(None, None, ak, ai, 0, True)
        one_pass2(ak, ai, bk, bi, 8, False)
        one_pass2(bk, bi, ak, ai, 16, False)
        one_pass2(ak, ai, bk, bi, 24, False)
        pltpu.sync_copy(bk.at[pl.ds(base, CHUNK)],
                        keys_hbm.at[row, pl.ds(base, CHUNK)])
        pltpu.sync_copy(bi.at[pl.ds(base, CHUNK)],
                        order_hbm.at[row, pl.ds(base, CHUNK)])
        return _

    lax.fori_loop(0, ROWS_PER_CORE, row_body2, 0, unroll=False)


def _sc_sort(xpad):
    mesh = plsc.VectorSubcoreMesh(core_axis_name="c", subcore_axis_name="s")
    f = pl.kernel(
        _sc_sort_body,
        mesh=mesh,
        compiler_params=pltpu.CompilerParams(needs_layout_passes=False),
        out_type=[
            jax.ShapeDtypeStruct((B, VPAD), jnp.int32),
            jax.ShapeDtypeStruct((B, VPAD), jnp.int32),
        ],
        scratch_types=[
            pltpu.VMEM_SHARED((VPAD,), jnp.int32),
            pltpu.VMEM_SHARED((VPAD,), jnp.int32),
            pltpu.VMEM_SHARED((VPAD,), jnp.int32),
            pltpu.VMEM_SHARED((VPAD,), jnp.int32),
            pltpu.VMEM_SHARED((2 * NTILES, 256), jnp.int32),
            pltpu.VMEM((CHUNK,), jnp.float32),
            pltpu.VMEM((CHUNK,), jnp.int32),
            pltpu.VMEM((CHUNK,), jnp.int32),
            pltpu.VMEM((NJ, 128), jnp.int32),
            pltpu.VMEM((512,), jnp.int32),
            pltpu.VMEM((512,), jnp.int32),
            pltpu.VMEM((2 * NTILES, 256), jnp.int32),
            pltpu.SemaphoreType.DMA,
        ],
    )
    return f(xpad)


ROWS_PER_BLOCK = 8


def _tail_kernel(keys_ref, order_ref, gum_ref, tri_ref, tri2_ref, out_ref):
    k = keys_ref[...]
    m = ~k
    u = jnp.where(m < 0, m ^ _MIN32, ~m)
    s = lax.bitcast_convert_type(u, jnp.float32)
    maxv = s[:, 0:1]
    e = jnp.exp(s - maxv)
    z = jnp.sum(e, axis=-1, keepdims=True)
    p = e / z
    p3 = p.reshape(ROWS_PER_BLOCK * (VPAD // 128), 128)
    within_excl = jnp.dot(p3, tri_ref[...],
                          preferred_element_type=jnp.float32)
    bsum = jnp.sum(p3, axis=-1).reshape(ROWS_PER_BLOCK, VPAD // 128)
    bcarry = jnp.dot(bsum, tri2_ref[...],
                     preferred_element_type=jnp.float32)
    excl = (within_excl.reshape(ROWS_PER_BLOCK, VPAD // 128, 128)
            + bcarry[:, :, None]).reshape(ROWS_PER_BLOCK, VPAD)
    keep = excl <= jnp.float32(TOP_P)
    score = jnp.where(keep, s + gum_ref[...], -jnp.inf)
    jstar = jnp.argmax(score, axis=-1, keepdims=True)
    cols = lax.broadcasted_iota(jnp.int32, (ROWS_PER_BLOCK, VPAD), 1)
    sel = jnp.where(cols == jstar, order_ref[...], jnp.int32(-1))
    out_ref[...] = jnp.max(sel, axis=-1, keepdims=True)


def _tail(keys, order, gumbel, tri, tri2):
    grid = (B // ROWS_PER_BLOCK,)
    return pl.pallas_call(
        _tail_kernel,
        grid=grid,
        in_specs=[
            pl.BlockSpec((ROWS_PER_BLOCK, VPAD), lambda i: (i, 0)),
            pl.BlockSpec((ROWS_PER_BLOCK, VPAD), lambda i: (i, 0)),
            pl.BlockSpec((ROWS_PER_BLOCK, VPAD), lambda i: (i, 0)),
            pl.BlockSpec((128, 128), lambda i: (0, 0)),
            pl.BlockSpec((VPAD // 128, VPAD // 128), lambda i: (0, 0)),
        ],
        out_specs=pl.BlockSpec((ROWS_PER_BLOCK, 1), lambda i: (i, 0)),
        out_shape=jax.ShapeDtypeStruct((B, 1), jnp.int32),
    )(keys, order, gumbel, tri, tri2)


def kernel(logits):
    x = logits / 1.0
    xpad = jnp.pad(x, ((0, 0), (0, VPAD - V)), constant_values=-jnp.inf)
    keys, order = _sc_sort(xpad)

    skey = jax.random.key(42)
    u = jax.random.uniform(skey, (B, V), dtype=jnp.float32,
                           minval=1e-20, maxval=1.0)
    gumbel = -jnp.log(-jnp.log(u))
    gumbel_p = jnp.pad(gumbel, ((0, 0), (0, VPAD - V)))

    tri = jnp.triu(jnp.ones((128, 128), jnp.float32), k=1)
    n2 = VPAD // 128
    tri2 = jnp.triu(jnp.ones((n2, n2), jnp.float32), k=1)

    return _tail(keys, order, gumbel_p, tri, tri2)
